# baseline (speedup 1.0000x reference)
import jax
import jax.numpy as jnp
from jax import lax
from jax.experimental import pallas as pl
from jax.experimental.pallas import tpu as pltpu
from jax.experimental.pallas import tpu_sc as plsc

B = 8192
D = 256
K = 8192

BB = 2048
KB = 1024
NB = B // BB
NK = K // KB


def _argmin_body(x_ref, cb_ref, ind_ref, loss_ref, rmax_ref, ridx_ref):
    b = pl.program_id(0)
    k = pl.program_id(1)

    @pl.when(k == 0)
    def _init():
        rmax_ref[...] = jnp.full((BB,), -jnp.inf, jnp.float32)
        ridx_ref[...] = jnp.zeros((BB,), jnp.int32)

    x = x_ref[...]
    cb = cb_ref[...]
    d = lax.dot_general(x, cb, (((1,), (1,)), ((), ())),
                        preferred_element_type=jnp.float32)
    x_sq = jnp.sum(x * x, axis=1, keepdims=True)
    cb_sq = jnp.sum(cb * cb, axis=1)[None, :]
    dist = -(x_sq - 2.0 * d + cb_sq)

    bm = jnp.max(dist, axis=1)
    iota = lax.broadcasted_iota(jnp.int32, (BB, KB), 1)
    bi = jnp.min(jnp.where(dist == bm[:, None], iota, K), axis=1)

    better = bm > rmax_ref[...]
    ridx_ref[...] = jnp.where(better, bi + k * KB, ridx_ref[...])
    rmax_ref[...] = jnp.where(better, bm, rmax_ref[...])

    @pl.when(k == NK - 1)
    def _emit():
        ind_ref[...] = ridx_ref[...]
        partial = -jnp.sum(rmax_ref[...]) / (B * D)

        @pl.when(b == 0)
        def _first():
            loss_ref[0, 0] = partial

        @pl.when(b > 0)
        def _rest():
            loss_ref[0, 0] = loss_ref[0, 0] + partial


def _argmin_tc(x, cb, interpret=False):
    return pl.pallas_call(
        _argmin_body,
        grid=(NB, NK),
        in_specs=[
            pl.BlockSpec((BB, D), lambda b, k: (b, 0)),
            pl.BlockSpec((KB, D), lambda b, k: (k, 0)),
        ],
        out_specs=[
            pl.BlockSpec((BB,), lambda b, k: (b,)),
            pl.BlockSpec((1, 1), lambda b, k: (0, 0),
                         memory_space=pltpu.SMEM),
        ],
        out_shape=[
            jax.ShapeDtypeStruct((B,), jnp.int32),
            jax.ShapeDtypeStruct((1, 1), jnp.float32),
        ],
        scratch_shapes=[
            pltpu.VMEM((BB,), jnp.float32),
            pltpu.VMEM((BB,), jnp.int32),
        ],
        interpret=interpret,
    )(x, cb)


_NC = 2
_NS = 16
_NW = _NC * _NS
_BPW = B // _NW
_CHUNK = 128
_NCH = _BPW // _CHUNK


def _gather_body(cb_hbm, idx_hbm, out_hbm, idx_v, rows_v, sem):
    wid = lax.axis_index("s") * _NC + lax.axis_index("c")
    base = wid * _BPW
    pltpu.sync_copy(idx_hbm.at[pl.ds(wid * _NCH, _NCH)], idx_v)
    copies = [
        pltpu.async_copy(cb_hbm.at[idx_v.at[j]],
                         rows_v.at[pl.ds(j * _CHUNK, _CHUNK)], sem)
        for j in range(_NCH)
    ]
    for c in copies:
        c.wait()
    pltpu.sync_copy(rows_v, out_hbm.at[pl.ds(base, _BPW)])


def _gather_sc(cb, ind):
    mesh = plsc.VectorSubcoreMesh(core_axis_name="c", subcore_axis_name="s")
    idx2d = ind.reshape(B // _CHUNK, _CHUNK)
    run = pl.kernel(
        _gather_body,
        out_type=jax.ShapeDtypeStruct((B, D), jnp.float32),
        mesh=mesh,
        scratch_types=[
            pltpu.VMEM((_NCH, _CHUNK), jnp.int32),
            pltpu.VMEM((_BPW, D), jnp.float32),
            pltpu.SemaphoreType.DMA,
        ],
    )
    return run(cb, idx2d)


def kernel(hidden_states, codebook):
    ind, loss = _argmin_tc(hidden_states, codebook)
    quantize = _gather_sc(codebook, ind)
    return quantize, ind, loss.reshape(())


# speedup vs baseline: 2.0470x; 2.0470x over previous
import jax
import jax.numpy as jnp
from jax import lax
from jax.experimental import pallas as pl
from jax.experimental.pallas import tpu as pltpu
from jax.experimental.pallas import tpu_sc as plsc

B = 8192
D = 256
K = 8192

BB = 2048
KB = 1024
NB = B // BB
NK = K // KB


def _argmin_body(x_ref, cb_ref, ind_ref, loss_ref,
                 rmax_ref, ridx_ref, x2_ref, bi_ref):
    b = pl.program_id(0)
    k = pl.program_id(1)

    @pl.when(k == 0)
    def _init():
        rmax_ref[...] = jnp.full((BB,), -jnp.inf, jnp.float32)
        ridx_ref[...] = jnp.zeros((BB,), jnp.int32)
        x = x_ref[...]
        x2_ref[...] = x + x

    cb = cb_ref[...]
    d = lax.dot_general(cb, x2_ref[...], (((1,), (1,)), ((), ())),
                        preferred_element_type=jnp.float32)
    cb_sq = jnp.sum(cb * cb, axis=1)[:, None]
    dist = d - cb_sq

    bm = jnp.max(dist, axis=0)
    eqf = jnp.where(dist == bm[None, :], 1.0, 0.0)
    ones_row = jnp.ones((1, KB), jnp.float32)
    iota_row = lax.broadcasted_iota(jnp.int32, (1, KB), 1).astype(jnp.float32)
    w = jnp.concatenate([ones_row, iota_row], axis=0)
    r = lax.dot_general(w, eqf, (((1,), (0,)), ((), ())),
                        preferred_element_type=jnp.float32)
    bi_ref[...] = (r[1] + 0.5).astype(jnp.int32)

    @pl.when(jnp.max(r[0]) > 1.5)
    def _ties():
        iota = lax.broadcasted_iota(jnp.int32, (KB, BB), 0)
        bi_ref[...] = jnp.min(jnp.where(dist == bm[None, :], iota, K),
                              axis=0)

    better = bm > rmax_ref[...]
    ridx_ref[...] = jnp.where(better, bi_ref[...] + k * KB, ridx_ref[...])
    rmax_ref[...] = jnp.where(better, bm, rmax_ref[...])

    @pl.when(k == NK - 1)
    def _emit():
        ind_ref[...] = ridx_ref[...]
        x = x_ref[...]
        x_sq_sum = jnp.sum(x * x)
        partial = (x_sq_sum - jnp.sum(rmax_ref[...])) / (B * D)

        @pl.when(b == 0)
        def _first():
            loss_ref[0, 0] = partial

        @pl.when(b > 0)
        def _rest():
            loss_ref[0, 0] = loss_ref[0, 0] + partial


def _argmin_tc(x, cb, interpret=False):
    return pl.pallas_call(
        _argmin_body,
        grid=(NB, NK),
        in_specs=[
            pl.BlockSpec((BB, D), lambda b, k: (b, 0)),
            pl.BlockSpec((KB, D), lambda b, k: (k, 0)),
        ],
        out_specs=[
            pl.BlockSpec((BB,), lambda b, k: (b,)),
            pl.BlockSpec((1, 1), lambda b, k: (0, 0),
                         memory_space=pltpu.SMEM),
        ],
        out_shape=[
            jax.ShapeDtypeStruct((B,), jnp.int32),
            jax.ShapeDtypeStruct((1, 1), jnp.float32),
        ],
        scratch_shapes=[
            pltpu.VMEM((BB,), jnp.float32),
            pltpu.VMEM((BB,), jnp.int32),
            pltpu.VMEM((BB, D), jnp.float32),
            pltpu.VMEM((BB,), jnp.int32),
        ],
        interpret=interpret,
    )(x, cb)


_NC = 2
_NS = 16
_NW = _NC * _NS
_BPW = B // _NW
_CHUNK = 128
_NCH = _BPW // _CHUNK


def _gather_body(cb_hbm, idx_hbm, out_hbm, idx_v, rows_v, sem):
    wid = lax.axis_index("s") * _NC + lax.axis_index("c")
    base = wid * _BPW
    pltpu.sync_copy(idx_hbm.at[pl.ds(wid * _NCH, _NCH)], idx_v)
    copies = [
        pltpu.async_copy(cb_hbm.at[idx_v.at[j]],
                         rows_v.at[pl.ds(j * _CHUNK, _CHUNK)], sem)
        for j in range(_NCH)
    ]
    for c in copies:
        c.wait()
    pltpu.sync_copy(rows_v, out_hbm.at[pl.ds(base, _BPW)])


def _gather_sc(cb, ind):
    mesh = plsc.VectorSubcoreMesh(core_axis_name="c", subcore_axis_name="s")
    idx2d = ind.reshape(B // _CHUNK, _CHUNK)
    run = pl.kernel(
        _gather_body,
        out_type=jax.ShapeDtypeStruct((B, D), jnp.float32),
        mesh=mesh,
        scratch_types=[
            pltpu.VMEM((_NCH, _CHUNK), jnp.int32),
            pltpu.VMEM((_BPW, D), jnp.float32),
            pltpu.SemaphoreType.DMA,
        ],
    )
    return run(cb, idx2d)


def kernel(hidden_states, codebook):
    ind, loss = _argmin_tc(hidden_states, codebook)
    quantize = _gather_sc(codebook, ind)
    return quantize, ind, loss.reshape(())
